# baseline (speedup 1.0000x reference)
import jax
import jax.numpy as jnp
from jax.experimental import pallas as pl


def _add_kernel(e_ref, p_ref, o_ref):
    o_ref[...] = e_ref[...] + p_ref[...][None, :, :]


def kernel(embed, pos_table):
    B, L, D = embed.shape
    BB = 8
    return pl.pallas_call(
        _add_kernel,
        grid=(B // BB,),
        in_specs=[
            pl.BlockSpec((BB, L, D), lambda i: (i, 0, 0)),
            pl.BlockSpec((L, D), lambda i: (0, 0)),
        ],
        out_specs=pl.BlockSpec((BB, L, D), lambda i: (i, 0, 0)),
        out_shape=jax.ShapeDtypeStruct((B, L, D), embed.dtype),
    )(embed, pos_table[:L])


# speedup vs baseline: 1.1057x; 1.1057x over previous
import jax
import jax.numpy as jnp
from jax.experimental import pallas as pl


def _add_kernel(e_ref, p_ref, o_ref):
    o_ref[...] = e_ref[...] + p_ref[...][None, :, :]


def kernel(embed, pos_table):
    B, L, D = embed.shape
    BB = 32
    return pl.pallas_call(
        _add_kernel,
        grid=(B // BB,),
        in_specs=[
            pl.BlockSpec((BB, L, D), lambda i: (i, 0, 0)),
            pl.BlockSpec((L, D), lambda i: (0, 0)),
        ],
        out_specs=pl.BlockSpec((BB, L, D), lambda i: (i, 0, 0)),
        out_shape=jax.ShapeDtypeStruct((B, L, D), embed.dtype),
    )(embed, pos_table[:L])
